# baseline (speedup 1.0000x reference)
import functools

import jax
import jax.numpy as jnp
from jax.experimental import pallas as pl
from jax.experimental.pallas import tpu as pltpu

_H_PAD = 56


def _mlp_body(xt_ref, w1_ref, b1_ref, w2_ref, b2_ref, o_ref):
    xb = xt_ref[...].astype(jnp.bfloat16)
    ht = jnp.dot(w1_ref[...], xb, preferred_element_type=jnp.float32)
    ht = jnp.tanh((ht + b1_ref[...]).astype(jnp.bfloat16))
    o_ref[...] = (
        jnp.dot(w2_ref[...], ht, preferred_element_type=jnp.float32)
        + b2_ref[...])


_RESIDENT = pl.BlockSpec(memory_space=pltpu.MemorySpace.VMEM)


@functools.partial(jax.jit, static_argnames=("block_b",))
def _forward(x, w1, b1, w2, b2, block_b=65536):
    B, S = x.shape
    H = w1.shape[0]
    A = w2.shape[0]

    b1 = b1.astype(jnp.float32).reshape(-1)
    b2 = b2.astype(jnp.float32).reshape(-1)

    w1p = jnp.zeros((_H_PAD, S), jnp.bfloat16).at[:H, :].set(
        w1.astype(jnp.bfloat16))
    b1p = jnp.zeros((_H_PAD, 1), jnp.float32).at[:H, 0].set(b1)
    w2p = jnp.zeros((A, _H_PAD), jnp.bfloat16).at[:, :H].set(
        w2.astype(jnp.bfloat16))
    b2p = b2[:, None]

    b_pad = -(-B // block_b) * block_b
    xt = jnp.zeros((S, b_pad), jnp.float32).at[:, :B].set(
        x.astype(jnp.float32).T)

    yt = pl.pallas_call(
        _mlp_body,
        out_shape=jax.ShapeDtypeStruct((A, b_pad), jnp.float32),
        grid=(b_pad // block_b,),
        in_specs=[
            pl.BlockSpec((S, block_b), lambda i: (0, i)),
            _RESIDENT, _RESIDENT, _RESIDENT, _RESIDENT,
        ],
        out_specs=pl.BlockSpec((A, block_b), lambda i: (0, i)),
        compiler_params=pltpu.CompilerParams(
            dimension_semantics=("parallel",)),
    )(xt, w1p, b1p, w2p, b2p)

    return yt[:, :B].T


def kernel(x, w1, b1, w2, b2):
    return _forward(x, w1, b1, w2, b2)


# speedup vs baseline: 1.4376x; 1.4376x over previous
import functools

import jax
import jax.numpy as jnp
from jax.experimental import pallas as pl
from jax.experimental.pallas import tpu as pltpu

_H_PAD = 56


def _mlp_body(xt_ref, w1_ref, b1_ref, w2_ref, b2_ref, o_ref):
    ht = jnp.dot(w1_ref[...], xt_ref[...], preferred_element_type=jnp.float32)
    ht = jnp.tanh(ht + b1_ref[...])
    o_ref[...] = (
        jnp.dot(w2_ref[...], ht, preferred_element_type=jnp.float32)
        + b2_ref[...])


_RESIDENT = pl.BlockSpec(memory_space=pltpu.MemorySpace.VMEM)


@functools.partial(jax.jit, static_argnames=("block_b",))
def _forward(x, w1, b1, w2, b2, block_b=131072):
    B, S = x.shape
    H = w1.shape[0]
    A = w2.shape[0]

    w1 = w1.astype(jnp.float32)
    b1 = b1.astype(jnp.float32).reshape(-1)
    w2 = w2.astype(jnp.float32)
    b2 = b2.astype(jnp.float32).reshape(-1)

    w1p = jnp.zeros((_H_PAD, S), jnp.float32).at[:H, :].set(w1)
    b1p = jnp.zeros((_H_PAD, 1), jnp.float32).at[:H, 0].set(b1)
    w2p = jnp.zeros((A, _H_PAD), jnp.float32).at[:, :H].set(w2)
    b2p = b2[:, None]

    b_pad = -(-B // block_b) * block_b
    xt = jnp.zeros((S, b_pad), jnp.float32).at[:, :B].set(x.T)

    yt = pl.pallas_call(
        _mlp_body,
        out_shape=jax.ShapeDtypeStruct((A, b_pad), jnp.float32),
        grid=(b_pad // block_b,),
        in_specs=[
            pl.BlockSpec((S, block_b), lambda i: (0, i)),
            _RESIDENT, _RESIDENT, _RESIDENT, _RESIDENT,
        ],
        out_specs=pl.BlockSpec((A, block_b), lambda i: (0, i)),
        compiler_params=pltpu.CompilerParams(
            dimension_semantics=("parallel",)),
    )(xt, w1p, b1p, w2p, b2p)

    return yt[:, :B].T


def kernel(x, w1, b1, w2, b2):
    return _forward(x, w1, b1, w2, b2)


# speedup vs baseline: 1.4577x; 1.0140x over previous
import functools

import jax
import jax.numpy as jnp
from jax.experimental import pallas as pl
from jax.experimental.pallas import tpu as pltpu

_H_PAD = 56


_CHUNKS = 4


def _mlp_body(xt_ref, w1_ref, b1_ref, w2_ref, b2_ref, o_ref):
    w = xt_ref.shape[1] // _CHUNKS
    for c in range(_CHUNKS):
        sl = pl.ds(c * w, w)
        ht = jnp.dot(w1_ref[...], xt_ref[:, sl],
                     preferred_element_type=jnp.float32)
        ht = jnp.tanh(ht + b1_ref[...])
        o_ref[:, sl] = (
            jnp.dot(w2_ref[...], ht, preferred_element_type=jnp.float32)
            + b2_ref[...])


_RESIDENT = pl.BlockSpec(memory_space=pltpu.MemorySpace.VMEM)


@functools.partial(jax.jit, static_argnames=("block_b",))
def _forward(x, w1, b1, w2, b2, block_b=262144):
    B, S = x.shape
    H = w1.shape[0]
    A = w2.shape[0]

    w1 = w1.astype(jnp.float32)
    b1 = b1.astype(jnp.float32).reshape(-1)
    w2 = w2.astype(jnp.float32)
    b2 = b2.astype(jnp.float32).reshape(-1)

    w1p = jnp.zeros((_H_PAD, S), jnp.float32).at[:H, :].set(w1)
    b1p = jnp.zeros((_H_PAD, 1), jnp.float32).at[:H, 0].set(b1)
    w2p = jnp.zeros((A, _H_PAD), jnp.float32).at[:, :H].set(w2)
    b2p = b2[:, None]

    b_pad = -(-B // block_b) * block_b
    xt = jnp.zeros((S, b_pad), jnp.float32).at[:, :B].set(x.T)

    yt = pl.pallas_call(
        _mlp_body,
        out_shape=jax.ShapeDtypeStruct((A, b_pad), jnp.float32),
        grid=(b_pad // block_b,),
        in_specs=[
            pl.BlockSpec((S, block_b), lambda i: (0, i)),
            _RESIDENT, _RESIDENT, _RESIDENT, _RESIDENT,
        ],
        out_specs=pl.BlockSpec((A, block_b), lambda i: (0, i)),
        compiler_params=pltpu.CompilerParams(
            dimension_semantics=("parallel",)),
    )(xt, w1p, b1p, w2p, b2p)

    return yt[:, :B].T


def kernel(x, w1, b1, w2, b2):
    return _forward(x, w1, b1, w2, b2)
